# baseline (speedup 1.0000x reference)
import functools

import jax
import jax.numpy as jnp
from jax import lax
from jax.experimental import pallas as pl
from jax.experimental.pallas import tpu as pltpu
from jax.experimental.pallas import tpu_sc as plsc

VOCAB = 1000000
EMBED_DIM = 32
DPAD = 128
DENSE_OUT = 350
BATCH = 16384
HIST = 20
BL = BATCH * HIST

NC = 2
NS = 16
NW = NC * NS

NSPLIT = 2
SPLIT = BL // NSPLIT
B_PER_W = SPLIT // NW
CHUNK = 256
NCHUNK = B_PER_W // CHUNK


def _gather_body(idx_hbm, table_hbm, out_hbm,
                 idx0, idx1, rows0, rows1, gsem0, gsem1, ssem0, ssem1):
    wid = lax.axis_index("s") * NC + lax.axis_index("c")
    base = wid * B_PER_W
    idx_v = (idx0, idx1)
    rows_v = (rows0, rows1)
    gsem = (gsem0, gsem1)
    ssem = (ssem0, ssem1)

    def start_gather(i, b):
        off = base + i * CHUNK
        pltpu.sync_copy(idx_hbm.at[pl.ds(off, CHUNK)], idx_v[b])
        return pltpu.async_copy(table_hbm.at[idx_v[b]], rows_v[b], gsem[b])

    def start_scatter(i, b):
        off = base + i * CHUNK
        return pltpu.async_copy(rows_v[b], out_hbm.at[pl.ds(off, CHUNK)],
                                ssem[b])

    start_gather(0, 0)

    def step(k, _):
        i0 = k * 2
        i1 = i0 + 1

        @pl.when(k > 0)
        def _():
            pltpu.make_async_copy(rows_v[1], out_hbm.at[pl.ds(0, CHUNK)],
                                  ssem[1]).wait()

        start_gather(i1, 1)
        pltpu.make_async_copy(table_hbm.at[idx_v[0]], rows_v[0],
                              gsem[0]).wait()
        start_scatter(i0, 0)

        @pl.when(k + 1 < NCHUNK // 2)
        def _():
            pltpu.make_async_copy(rows_v[0], out_hbm.at[pl.ds(0, CHUNK)],
                                  ssem[0]).wait()
            start_gather(i0 + 2, 0)

        pltpu.make_async_copy(table_hbm.at[idx_v[1]], rows_v[1],
                              gsem[1]).wait()
        start_scatter(i1, 1)
        return 0

    lax.fori_loop(0, NCHUNK // 2, step, 0)
    pltpu.make_async_copy(rows_v[0], out_hbm.at[pl.ds(0, CHUNK)],
                          ssem[0]).wait()
    pltpu.make_async_copy(rows_v[1], out_hbm.at[pl.ds(0, CHUNK)],
                          ssem[1]).wait()


@functools.cache
def _sc_gather():
    return pl.kernel(
        _gather_body,
        out_type=jax.ShapeDtypeStruct((SPLIT, DPAD), jnp.float32),
        mesh=plsc.VectorSubcoreMesh(
            core_axis_name="c", subcore_axis_name="s",
            num_cores=NC, num_subcores=NS,
        ),
        scratch_types=[
            pltpu.VMEM((CHUNK,), jnp.int32),
            pltpu.VMEM((CHUNK,), jnp.int32),
            pltpu.VMEM((CHUNK, DPAD), jnp.float32),
            pltpu.VMEM((CHUNK, DPAD), jnp.float32),
            pltpu.SemaphoreType.DMA,
            pltpu.SemaphoreType.DMA,
            pltpu.SemaphoreType.DMA,
            pltpu.SemaphoreType.DMA,
        ],
        compiler_params=pltpu.CompilerParams(use_tc_tiling_on_sc=False),
    )


BT = 32768


def _tp_body(xt_ref, o_ref):
    xt = jnp.transpose(xt_ref[...], (1, 0))
    o_ref[...] = jnp.concatenate(
        [xt, jnp.zeros((BT, DPAD - EMBED_DIM), jnp.float32)], axis=1)


def _tc_padtable(tableT):
    return pl.pallas_call(
        _tp_body,
        grid=(pl.cdiv(VOCAB, BT),),
        in_specs=[pl.BlockSpec((EMBED_DIM, BT), lambda i: (0, i))],
        out_specs=pl.BlockSpec((BT, DPAD), lambda i: (i, 0)),
        out_shape=jax.ShapeDtypeStruct((VOCAB, DPAD), jnp.float32),
    )(tableT)


BB = 8192
LS = HIST // NSPLIT


def _mm_body(x_ref, w_ref, b_ref, o_ref):
    x = x_ref[0]
    w = w_ref[...]
    y = lax.dot_general(w, x, (((0,), (1,)), ((), ())),
                        preferred_element_type=jnp.float32)
    o_ref[0] = y + b_ref[...]


def _mm_chain_body(x_ref, w_ref, b_ref, prev_ref, o_ref):
    del prev_ref
    _mm_body(x_ref, w_ref, b_ref, o_ref)


OUT_SHAPE = jax.ShapeDtypeStruct((HIST, DENSE_OUT, BATCH), jnp.float32)


def _tc_matmul(emb3, w_pad, b2, prev, l_off):
    in_specs = [
        pl.BlockSpec((1, BB, DPAD), lambda l, i: (l, i, 0)),
        pl.BlockSpec((DPAD, DENSE_OUT), lambda l, i: (0, 0)),
        pl.BlockSpec((DENSE_OUT, 1), lambda l, i: (0, 0)),
    ]
    args = [emb3, w_pad, b2]
    aliases = {}
    if prev is not None:
        in_specs.append(pl.BlockSpec((1, 8, 128), lambda l, i: (0, 0, 0)))
        args.append(prev)
        aliases = {3: 0}
    return pl.pallas_call(
        _mm_body if prev is None else _mm_chain_body,
        grid=(LS, BATCH // BB),
        in_specs=in_specs,
        out_specs=pl.BlockSpec((1, DENSE_OUT, BB),
                               lambda l, i, o=l_off: (l + o, 0, i)),
        out_shape=OUT_SHAPE,
        input_output_aliases=aliases,
    )(*args)


def kernel(inputs, table, W, b):
    idx = jnp.transpose(inputs).reshape(BL)
    table_pad = _tc_padtable(jnp.transpose(table))
    w_pad = jnp.pad(W, ((0, DPAD - EMBED_DIM), (0, 0)))
    b2 = b.reshape(DENSE_OUT, 1)
    embs = [_sc_gather()(idx[q * SPLIT:(q + 1) * SPLIT], table_pad)
            for q in range(NSPLIT)]
    out = None
    for q in range(NSPLIT):
        emb3 = embs[q].reshape(LS, BATCH, DPAD)
        out = _tc_matmul(emb3, w_pad, b2, out, q * LS)
    return out.transpose(2, 0, 1)


# speedup vs baseline: 1.0108x; 1.0108x over previous
import functools

import jax
import jax.numpy as jnp
from jax import lax
from jax.experimental import pallas as pl
from jax.experimental.pallas import tpu as pltpu
from jax.experimental.pallas import tpu_sc as plsc

VOCAB = 1000000
EMBED_DIM = 32
DPAD = 128
DENSE_OUT = 350
BATCH = 16384
HIST = 20
BL = BATCH * HIST

NC = 2
NS = 16
NW = NC * NS

NSPLIT = 1
SPLIT = BL // NSPLIT
B_PER_W = SPLIT // NW
CHUNK = 256
NCHUNK = B_PER_W // CHUNK


def _gather_body(idx_hbm, table_hbm, out_hbm,
                 idx0, idx1, rows0, rows1, gsem0, gsem1, ssem0, ssem1):
    wid = lax.axis_index("s") * NC + lax.axis_index("c")
    base = wid * B_PER_W
    idx_v = (idx0, idx1)
    rows_v = (rows0, rows1)
    gsem = (gsem0, gsem1)
    ssem = (ssem0, ssem1)

    def start_gather(i, b):
        off = base + i * CHUNK
        pltpu.sync_copy(idx_hbm.at[pl.ds(off, CHUNK)], idx_v[b])
        return pltpu.async_copy(table_hbm.at[idx_v[b]], rows_v[b], gsem[b])

    def start_scatter(i, b):
        off = base + i * CHUNK
        return pltpu.async_copy(rows_v[b], out_hbm.at[pl.ds(off, CHUNK)],
                                ssem[b])

    start_gather(0, 0)

    def step(k, _):
        i0 = k * 2
        i1 = i0 + 1

        @pl.when(k > 0)
        def _():
            pltpu.make_async_copy(rows_v[1], out_hbm.at[pl.ds(0, CHUNK)],
                                  ssem[1]).wait()

        start_gather(i1, 1)
        pltpu.make_async_copy(table_hbm.at[idx_v[0]], rows_v[0],
                              gsem[0]).wait()
        start_scatter(i0, 0)

        @pl.when(k + 1 < NCHUNK // 2)
        def _():
            pltpu.make_async_copy(rows_v[0], out_hbm.at[pl.ds(0, CHUNK)],
                                  ssem[0]).wait()
            start_gather(i0 + 2, 0)

        pltpu.make_async_copy(table_hbm.at[idx_v[1]], rows_v[1],
                              gsem[1]).wait()
        start_scatter(i1, 1)
        return 0

    lax.fori_loop(0, NCHUNK // 2, step, 0)
    pltpu.make_async_copy(rows_v[0], out_hbm.at[pl.ds(0, CHUNK)],
                          ssem[0]).wait()
    pltpu.make_async_copy(rows_v[1], out_hbm.at[pl.ds(0, CHUNK)],
                          ssem[1]).wait()


@functools.cache
def _sc_gather():
    return pl.kernel(
        _gather_body,
        out_type=jax.ShapeDtypeStruct((SPLIT, DPAD), jnp.float32),
        mesh=plsc.VectorSubcoreMesh(
            core_axis_name="c", subcore_axis_name="s",
            num_cores=NC, num_subcores=NS,
        ),
        scratch_types=[
            pltpu.VMEM((CHUNK,), jnp.int32),
            pltpu.VMEM((CHUNK,), jnp.int32),
            pltpu.VMEM((CHUNK, DPAD), jnp.float32),
            pltpu.VMEM((CHUNK, DPAD), jnp.float32),
            pltpu.SemaphoreType.DMA,
            pltpu.SemaphoreType.DMA,
            pltpu.SemaphoreType.DMA,
            pltpu.SemaphoreType.DMA,
        ],
        compiler_params=pltpu.CompilerParams(use_tc_tiling_on_sc=False),
    )


BT = 32768


def _tp_body(xt_ref, o_ref):
    xt = jnp.transpose(xt_ref[...], (1, 0))
    o_ref[...] = jnp.concatenate(
        [xt, jnp.zeros((BT, DPAD - EMBED_DIM), jnp.float32)], axis=1)


def _tc_padtable(tableT):
    return pl.pallas_call(
        _tp_body,
        grid=(pl.cdiv(VOCAB, BT),),
        in_specs=[pl.BlockSpec((EMBED_DIM, BT), lambda i: (0, i))],
        out_specs=pl.BlockSpec((BT, DPAD), lambda i: (i, 0)),
        out_shape=jax.ShapeDtypeStruct((VOCAB, DPAD), jnp.float32),
    )(tableT)


BB = 16384
LS = HIST // NSPLIT


def _mm_body(x_ref, w_ref, b_ref, o_ref):
    x = x_ref[0]
    w = w_ref[...]
    y = lax.dot_general(w, x, (((0,), (1,)), ((), ())),
                        preferred_element_type=jnp.float32)
    o_ref[0] = y + b_ref[...]


def _mm_chain_body(x_ref, w_ref, b_ref, prev_ref, o_ref):
    del prev_ref
    _mm_body(x_ref, w_ref, b_ref, o_ref)


OUT_SHAPE = jax.ShapeDtypeStruct((HIST, DENSE_OUT, BATCH), jnp.float32)


def _tc_matmul(emb3, w_pad, b2, prev, l_off):
    in_specs = [
        pl.BlockSpec((1, BB, DPAD), lambda l, i: (l, i, 0)),
        pl.BlockSpec((DPAD, DENSE_OUT), lambda l, i: (0, 0)),
        pl.BlockSpec((DENSE_OUT, 1), lambda l, i: (0, 0)),
    ]
    args = [emb3, w_pad, b2]
    aliases = {}
    if prev is not None:
        in_specs.append(pl.BlockSpec((1, 8, 128), lambda l, i: (0, 0, 0)))
        args.append(prev)
        aliases = {3: 0}
    return pl.pallas_call(
        _mm_body if prev is None else _mm_chain_body,
        grid=(LS, BATCH // BB),
        in_specs=in_specs,
        out_specs=pl.BlockSpec((1, DENSE_OUT, BB),
                               lambda l, i, o=l_off: (l + o, 0, i)),
        out_shape=OUT_SHAPE,
        input_output_aliases=aliases,
        compiler_params=pltpu.CompilerParams(
            vmem_limit_bytes=128 * 1024 * 1024),
    )(*args)


def kernel(inputs, table, W, b):
    idx = jnp.transpose(inputs).reshape(BL)
    table_pad = _tc_padtable(jnp.transpose(table))
    w_pad = jnp.pad(W, ((0, DPAD - EMBED_DIM), (0, 0)))
    b2 = b.reshape(DENSE_OUT, 1)
    embs = [_sc_gather()(idx[q * SPLIT:(q + 1) * SPLIT], table_pad)
            for q in range(NSPLIT)]
    out = None
    for q in range(NSPLIT):
        emb3 = embs[q].reshape(LS, BATCH, DPAD)
        out = _tc_matmul(emb3, w_pad, b2, out, q * LS)
    return out.transpose(2, 0, 1)
